# baseline (speedup 1.0000x reference)
import jax
import jax.numpy as jnp
from jax.experimental import pallas as pl
from jax.experimental.pallas import tpu as pltpu

NUM_CLASSES = 1000
CTX_LEN = 4
EMB_DIM = 64
ROW = (CTX_LEN + 1) * EMB_DIM
CTX_FLAT = CTX_LEN * EMB_DIM
N_CHUNKS = 25
CHUNK = NUM_CLASSES // N_CHUNKS


def _body(ctx_hbm, emb_hbm, out_hbm, ctx_v, blk, emb_v, in_sem, out_sem):
    ctx_in = pltpu.make_async_copy(ctx_hbm, ctx_v, in_sem)
    emb_in = pltpu.make_async_copy(emb_hbm, emb_v, in_sem)
    ctx_in.start()
    emb_in.start()

    out_copies = [
        pltpu.make_async_copy(
            blk.at[pl.ds(i * CHUNK, CHUNK), :],
            out_hbm.at[pl.ds(i * CHUNK, CHUNK), :],
            out_sem,
        )
        for i in range(N_CHUNKS)
    ]

    ctx_in.wait()
    parts = [
        jnp.broadcast_to(ctx_v[j : j + 1, :], (CHUNK, EMB_DIM))
        for j in range(CTX_LEN)
    ]
    for i in range(N_CHUNKS):
        if i == 0:
            emb_in.wait()
        blk[pl.ds(i * CHUNK, CHUNK), :] = jnp.concatenate(
            parts + [emb_v[pl.ds(i * CHUNK, CHUNK), :]], axis=1
        )
        out_copies[i].start()

    for c in out_copies:
        c.wait()


def kernel(context, emb_table):
    out2d = pl.pallas_call(
        _body,
        in_specs=[
            pl.BlockSpec(memory_space=pltpu.MemorySpace.HBM),
            pl.BlockSpec(memory_space=pltpu.MemorySpace.HBM),
        ],
        out_specs=pl.BlockSpec(memory_space=pltpu.MemorySpace.HBM),
        out_shape=jax.ShapeDtypeStruct((NUM_CLASSES, ROW), jnp.float32),
        compiler_params=pltpu.CompilerParams(
            skip_device_barrier=True,
            disable_bounds_checks=True,
            disable_semaphore_checks=True,
        ),
        scratch_shapes=[
            pltpu.VMEM((CTX_LEN, EMB_DIM), jnp.float32),
            pltpu.VMEM((NUM_CLASSES, ROW), jnp.float32),
            pltpu.VMEM((NUM_CLASSES, EMB_DIM), jnp.float32),
            pltpu.SemaphoreType.DMA,
            pltpu.SemaphoreType.DMA,
        ],
    )(context, emb_table)
    return out2d.reshape(NUM_CLASSES, CTX_LEN + 1, EMB_DIM)


# speedup vs baseline: 1.0245x; 1.0245x over previous
import jax
import jax.numpy as jnp
from jax.experimental import pallas as pl
from jax.experimental.pallas import tpu as pltpu

NUM_CLASSES = 1000
CTX_LEN = 4
EMB_DIM = 64
ROW = (CTX_LEN + 1) * EMB_DIM
CTX_FLAT = CTX_LEN * EMB_DIM
N_CHUNKS = 5
CHUNK = NUM_CLASSES // N_CHUNKS


def _body(ctx_hbm, emb_hbm, out_hbm, ctx_v, blk, emb_v, in_sem, emb_sems, out_sem):
    ctx_in = pltpu.make_async_copy(ctx_hbm, ctx_v, in_sem)
    ctx_in.start()
    emb_ins = [
        pltpu.make_async_copy(
            emb_hbm.at[pl.ds(i * CHUNK, CHUNK), :],
            emb_v.at[pl.ds(i * CHUNK, CHUNK), :],
            emb_sems.at[i],
        )
        for i in range(N_CHUNKS)
    ]
    for c in emb_ins:
        c.start()

    out_copies = [
        pltpu.make_async_copy(
            blk.at[pl.ds(i * CHUNK, CHUNK), :],
            out_hbm.at[pl.ds(i * CHUNK, CHUNK), :],
            out_sem,
        )
        for i in range(N_CHUNKS)
    ]

    ctx_in.wait()
    parts = [
        jnp.broadcast_to(ctx_v[j : j + 1, :], (CHUNK, EMB_DIM))
        for j in range(CTX_LEN)
    ]
    for i in range(N_CHUNKS):
        emb_ins[i].wait()
        blk[pl.ds(i * CHUNK, CHUNK), :] = jnp.concatenate(
            parts + [emb_v[pl.ds(i * CHUNK, CHUNK), :]], axis=1
        )
        out_copies[i].start()

    for c in out_copies:
        c.wait()


def kernel(context, emb_table):
    out2d = pl.pallas_call(
        _body,
        in_specs=[
            pl.BlockSpec(memory_space=pltpu.MemorySpace.HBM),
            pl.BlockSpec(memory_space=pltpu.MemorySpace.HBM),
        ],
        out_specs=pl.BlockSpec(memory_space=pltpu.MemorySpace.HBM),
        out_shape=jax.ShapeDtypeStruct((NUM_CLASSES, ROW), jnp.float32),
        compiler_params=pltpu.CompilerParams(
            skip_device_barrier=True,
            disable_bounds_checks=True,
            disable_semaphore_checks=True,
        ),
        scratch_shapes=[
            pltpu.VMEM((CTX_LEN, EMB_DIM), jnp.float32),
            pltpu.VMEM((NUM_CLASSES, ROW), jnp.float32),
            pltpu.VMEM((NUM_CLASSES, EMB_DIM), jnp.float32),
            pltpu.SemaphoreType.DMA,
            pltpu.SemaphoreType.DMA((N_CHUNKS,)),
            pltpu.SemaphoreType.DMA,
        ],
    )(context, emb_table)
    return out2d.reshape(NUM_CLASSES, CTX_LEN + 1, EMB_DIM)
